# baseline (speedup 1.0000x reference)
import functools

import jax
import jax.numpy as jnp
from jax import lax
from jax.experimental import pallas as pl
from jax.experimental.pallas import tpu as pltpu
from jax.experimental.pallas import tpu_sc as plsc

_MASK_RATIO = 0.4

_SC_CORES = 2
_SC_SUBCORES = 16
_LANES = 16


def _sc_mask_body(len_keep, l_total, nvars, noise_hbm, out_hbm, col_v, mask_v):
    nchunks = l_total // _LANES
    kvec = jnp.full((_LANES,), len_keep, jnp.int32)
    lane = lax.iota(jnp.int32, _LANES)
    wid = lax.axis_index("s") * _SC_CORES + lax.axis_index("c")

    def lane_gather(y, src):
        return lax.gather(
            y, src[:, None],
            lax.GatherDimensionNumbers(
                offset_dims=(), collapsed_slice_dims=(0,),
                start_index_map=(0,)),
            slice_sizes=(1,),
            mode=lax.GatherScatterMode.PROMISE_IN_BOUNDS)

    def all_sum(x):
        y = x
        for sh in (1, 2, 4, 8):
            y = y + lane_gather(y, jnp.bitwise_xor(lane, sh))
        return y

    def count_below(i, cand):
        def chunk(j, acc):
            off = pl.multiple_of(j * _LANES, _LANES)
            bits = col_v[i, pl.ds(off, _LANES)]
            return acc + jnp.where(bits < cand, 1, 0)
        acc = lax.fori_loop(0, nchunks, chunk,
                            jnp.zeros((_LANES,), jnp.int32))
        return all_sum(acc)

    def prefix_incl(x):
        y = x
        for sh in (1, 2, 4, 8):
            g = lane_gather(y, jnp.maximum(lane - sh, 0))
            y = y + jnp.where(lane >= sh, g, 0)
        return y

    pltpu.sync_copy(noise_hbm.at[wid], col_v)

    def col_body(i, _):
        def bit_iter(bi, t):
            cand = t + (jnp.int32(1) << (jnp.int32(29) - bi))
            return jnp.where(count_below(i, cand) < kvec, cand, t)
        t = lax.fori_loop(0, 30, bit_iter, jnp.zeros((_LANES,), jnp.int32))

        n_tie_keep = kvec - count_below(i, t)

        def tie_chunk(j, seq):
            off = pl.multiple_of(j * _LANES, _LANES)
            bits = col_v[i, pl.ds(off, _LANES)]
            below = bits < t
            eq = bits == t
            eqi = jnp.where(eq, 1, 0)
            incl = prefix_incl(eqi)
            excl = incl - eqi
            keep = below | (eq & ((seq + excl) < n_tie_keep))
            mask_v[i, pl.ds(off, _LANES)] = jnp.where(keep, 1.0, 0.0).astype(
                jnp.float32)
            return seq + lane_gather(incl, jnp.full((_LANES,), _LANES - 1,
                                                    jnp.int32))
        lax.fori_loop(0, nchunks, tie_chunk, jnp.zeros((_LANES,), jnp.int32))
        return 0

    lax.fori_loop(0, nvars, col_body, 0)
    pltpu.sync_copy(mask_v, out_hbm.at[wid])


def _apply_kernel(xb_ref, mask_ref, out_ref):
    m = mask_ref[0]
    mt = jnp.transpose(m)
    out_ref[...] = xb_ref[...] * mt[None, :, :, None]


@jax.jit
def kernel(xb, noise):
    bs, L, nvars, D = xb.shape
    len_keep = int(L * (1 - _MASK_RATIO))

    noise_r = jnp.transpose(noise, (0, 2, 1))

    mesh = plsc.VectorSubcoreMesh(core_axis_name="c", subcore_axis_name="s")
    sc_mask = functools.partial(
        pl.kernel,
        mesh=mesh,
        out_type=jax.ShapeDtypeStruct((bs, nvars, L), jnp.float32),
        scratch_types=[
            pltpu.VMEM((nvars, L), jnp.int32),
            pltpu.VMEM((nvars, L), jnp.float32),
        ],
    )(functools.partial(_sc_mask_body, len_keep, L, nvars))
    mask = sc_mask(lax.bitcast_convert_type(noise_r, jnp.int32))

    lb = 128
    grid = (bs, L // lb)
    out = pl.pallas_call(
        _apply_kernel,
        grid=grid,
        in_specs=[
            pl.BlockSpec((1, lb, nvars, D), lambda b, l: (b, l, 0, 0)),
            pl.BlockSpec((1, nvars, lb), lambda b, l: (b, 0, l)),
        ],
        out_specs=pl.BlockSpec((1, lb, nvars, D), lambda b, l: (b, l, 0, 0)),
        out_shape=jax.ShapeDtypeStruct((bs, L, nvars, D), jnp.float32),
    )(xb, mask)
    return out


# speedup vs baseline: 1.1242x; 1.1242x over previous
import functools

import jax
import jax.numpy as jnp
from jax import lax
from jax.experimental import pallas as pl
from jax.experimental.pallas import tpu as pltpu
from jax.experimental.pallas import tpu_sc as plsc

_MASK_RATIO = 0.4

_SC_CORES = 2
_SC_SUBCORES = 16
_LANES = 16


def _sc_mask_body(len_keep, l_total, nvars, noise_hbm, out_hbm, col_v, mask_v):
    nchunks = l_total // _LANES
    kvec = jnp.full((_LANES,), len_keep, jnp.int32)
    lane = lax.iota(jnp.int32, _LANES)
    wid = lax.axis_index("s") * _SC_CORES + lax.axis_index("c")

    def lane_gather(y, src):
        return lax.gather(
            y, src[:, None],
            lax.GatherDimensionNumbers(
                offset_dims=(), collapsed_slice_dims=(0,),
                start_index_map=(0,)),
            slice_sizes=(1,),
            mode=lax.GatherScatterMode.PROMISE_IN_BOUNDS)

    def all_sum(x):
        y = x
        for sh in (1, 2, 4, 8):
            y = y + lane_gather(y, jnp.bitwise_xor(lane, sh))
        return y

    def count_below(i, cand):
        def chunk(j, acc):
            off = pl.multiple_of(j * _LANES, _LANES)
            bits = col_v[i, pl.ds(off, _LANES)]
            return acc + jnp.where(bits < cand, 1, 0)
        acc = lax.fori_loop(0, nchunks, chunk,
                            jnp.zeros((_LANES,), jnp.int32))
        return all_sum(acc)

    def prefix_incl(x):
        y = x
        for sh in (1, 2, 4, 8):
            g = lane_gather(y, jnp.maximum(lane - sh, 0))
            y = y + jnp.where(lane >= sh, g, 0)
        return y

    pltpu.sync_copy(noise_hbm.at[wid], col_v)

    def col_body(i, _):
        def bit_iter(bi, t):
            cand = t + (jnp.int32(1) << (jnp.int32(29) - bi))
            return jnp.where(count_below(i, cand) < kvec, cand, t)
        t = lax.fori_loop(0, 30, bit_iter, jnp.zeros((_LANES,), jnp.int32))

        n_tie_keep = kvec - count_below(i, t)

        def tie_chunk(j, seq):
            off = pl.multiple_of(j * _LANES, _LANES)
            bits = col_v[i, pl.ds(off, _LANES)]
            below = bits < t
            eq = bits == t
            eqi = jnp.where(eq, 1, 0)
            incl = prefix_incl(eqi)
            excl = incl - eqi
            keep = below | (eq & ((seq + excl) < n_tie_keep))
            mask_v[i, pl.ds(off, _LANES)] = jnp.where(keep, 1.0, 0.0).astype(
                jnp.float32)
            return seq + lane_gather(incl, jnp.full((_LANES,), _LANES - 1,
                                                    jnp.int32))
        lax.fori_loop(0, nchunks, tie_chunk, jnp.zeros((_LANES,), jnp.int32))
        return 0

    lax.fori_loop(0, nvars, col_body, 0)
    pltpu.sync_copy(mask_v, out_hbm.at[wid])


def _apply_kernel(xb_ref, mask_ref, out_ref):
    m = mask_ref[...]
    mt = jnp.transpose(m, (0, 2, 1))
    out_ref[...] = xb_ref[...] * mt[:, :, :, None]


@jax.jit
def kernel(xb, noise):
    bs, L, nvars, D = xb.shape
    len_keep = int(L * (1 - _MASK_RATIO))

    noise_r = jnp.transpose(noise, (0, 2, 1))

    mesh = plsc.VectorSubcoreMesh(core_axis_name="c", subcore_axis_name="s")
    sc_mask = functools.partial(
        pl.kernel,
        mesh=mesh,
        out_type=jax.ShapeDtypeStruct((bs, nvars, L), jnp.float32),
        scratch_types=[
            pltpu.VMEM((nvars, L), jnp.int32),
            pltpu.VMEM((nvars, L), jnp.float32),
        ],
    )(functools.partial(_sc_mask_body, len_keep, L, nvars))
    mask = sc_mask(lax.bitcast_convert_type(noise_r, jnp.int32))

    bb = 2
    grid = (bs // bb,)
    out = pl.pallas_call(
        _apply_kernel,
        grid=grid,
        in_specs=[
            pl.BlockSpec((bb, L, nvars, D), lambda b: (b, 0, 0, 0)),
            pl.BlockSpec((bb, nvars, L), lambda b: (b, 0, 0)),
        ],
        out_specs=pl.BlockSpec((bb, L, nvars, D), lambda b: (b, 0, 0, 0)),
        out_shape=jax.ShapeDtypeStruct((bs, L, nvars, D), jnp.float32),
    )(xb, mask)
    return out


# speedup vs baseline: 1.1259x; 1.0016x over previous
import functools

import jax
import jax.numpy as jnp
from jax import lax
from jax.experimental import pallas as pl
from jax.experimental.pallas import tpu as pltpu
from jax.experimental.pallas import tpu_sc as plsc

_MASK_RATIO = 0.4

_SC_CORES = 2
_SC_SUBCORES = 16
_LANES = 16


def _sc_mask_body(len_keep, l_total, nvars, noise_hbm, out_hbm, col_v, mask_v):
    nchunks = l_total // _LANES
    kvec = jnp.full((_LANES,), len_keep, jnp.int32)
    lane = lax.iota(jnp.int32, _LANES)
    wid = lax.axis_index("s") * _SC_CORES + lax.axis_index("c")

    def lane_gather(y, src):
        return lax.gather(
            y, src[:, None],
            lax.GatherDimensionNumbers(
                offset_dims=(), collapsed_slice_dims=(0,),
                start_index_map=(0,)),
            slice_sizes=(1,),
            mode=lax.GatherScatterMode.PROMISE_IN_BOUNDS)

    def all_sum(x):
        y = x
        for sh in (1, 2, 4, 8):
            y = y + lane_gather(y, jnp.bitwise_xor(lane, sh))
        return y

    def count_below(i, cand):
        def chunk(j, acc):
            off = pl.multiple_of(j * _LANES, _LANES)
            bits = col_v[i, pl.ds(off, _LANES)]
            return acc + jnp.where(bits < cand, 1, 0)
        acc = lax.fori_loop(0, nchunks, chunk,
                            jnp.zeros((_LANES,), jnp.int32))
        return all_sum(acc)

    def prefix_incl(x):
        y = x
        for sh in (1, 2, 4, 8):
            g = lane_gather(y, jnp.maximum(lane - sh, 0))
            y = y + jnp.where(lane >= sh, g, 0)
        return y

    pltpu.sync_copy(noise_hbm.at[wid], col_v)

    def col_body(i, _):
        def bit_iter(bi, t):
            cand = t + (jnp.int32(1) << (jnp.int32(29) - bi))
            return jnp.where(count_below(i, cand) < kvec, cand, t)
        t = lax.fori_loop(0, 30, bit_iter, jnp.zeros((_LANES,), jnp.int32))

        n_tie_keep = kvec - count_below(i, t)

        def tie_chunk(j, seq):
            off = pl.multiple_of(j * _LANES, _LANES)
            bits = col_v[i, pl.ds(off, _LANES)]
            below = bits < t
            eq = bits == t
            eqi = jnp.where(eq, 1, 0)
            incl = prefix_incl(eqi)
            excl = incl - eqi
            keep = below | (eq & ((seq + excl) < n_tie_keep))
            mask_v[i, pl.ds(off, _LANES)] = jnp.where(keep, 1.0, 0.0).astype(
                jnp.float32)
            return seq + lane_gather(incl, jnp.full((_LANES,), _LANES - 1,
                                                    jnp.int32))
        lax.fori_loop(0, nchunks, tie_chunk, jnp.zeros((_LANES,), jnp.int32))
        return 0

    lax.fori_loop(0, nvars, col_body, 0)
    pltpu.sync_copy(mask_v, out_hbm.at[wid])


def _apply_kernel(xb_ref, mask_ref, out_ref):
    m = mask_ref[...]
    mt = jnp.transpose(m, (0, 2, 1))
    out_ref[...] = xb_ref[...] * mt[:, :, :, None]


@jax.jit
def kernel(xb, noise):
    bs, L, nvars, D = xb.shape
    len_keep = int(L * (1 - _MASK_RATIO))

    noise_r = jnp.transpose(noise, (0, 2, 1))

    mesh = plsc.VectorSubcoreMesh(core_axis_name="c", subcore_axis_name="s")
    sc_mask = functools.partial(
        pl.kernel,
        mesh=mesh,
        out_type=jax.ShapeDtypeStruct((bs, nvars, L), jnp.float32),
        scratch_types=[
            pltpu.VMEM((nvars, L), jnp.int32),
            pltpu.VMEM((nvars, L), jnp.float32),
        ],
    )(functools.partial(_sc_mask_body, len_keep, L, nvars))
    mask = sc_mask(lax.bitcast_convert_type(noise_r, jnp.int32))

    bb = 2
    grid = (bs // bb,)
    out = pl.pallas_call(
        _apply_kernel,
        grid=grid,
        compiler_params=pltpu.CompilerParams(
            dimension_semantics=("parallel",)),
        in_specs=[
            pl.BlockSpec((bb, L, nvars, D), lambda b: (b, 0, 0, 0)),
            pl.BlockSpec((bb, nvars, L), lambda b: (b, 0, 0)),
        ],
        out_specs=pl.BlockSpec((bb, L, nvars, D), lambda b: (b, 0, 0, 0)),
        out_shape=jax.ShapeDtypeStruct((bs, L, nvars, D), jnp.float32),
    )(xb, mask)
    return out


# speedup vs baseline: 1.1260x; 1.0001x over previous
import functools

import jax
import jax.numpy as jnp
from jax import lax
from jax.experimental import pallas as pl
from jax.experimental.pallas import tpu as pltpu
from jax.experimental.pallas import tpu_sc as plsc

_MASK_RATIO = 0.4

_SC_CORES = 2
_SC_SUBCORES = 16
_LANES = 16


def _sc_mask_body(len_keep, l_total, nvars, noise_hbm, out_hbm, col_v, mask_v):
    nchunks = l_total // _LANES
    kvec = jnp.full((_LANES,), len_keep, jnp.int32)
    lane = lax.iota(jnp.int32, _LANES)
    wid = lax.axis_index("s") * _SC_CORES + lax.axis_index("c")

    def lane_gather(y, src):
        return lax.gather(
            y, src[:, None],
            lax.GatherDimensionNumbers(
                offset_dims=(), collapsed_slice_dims=(0,),
                start_index_map=(0,)),
            slice_sizes=(1,),
            mode=lax.GatherScatterMode.PROMISE_IN_BOUNDS)

    def all_sum(x):
        y = x
        for sh in (1, 2, 4, 8):
            y = y + lane_gather(y, jnp.bitwise_xor(lane, sh))
        return y

    def count_below(i, cand):
        def chunk(j, acc):
            off = pl.multiple_of(j * _LANES, _LANES)
            bits = col_v[i, pl.ds(off, _LANES)]
            return acc + jnp.where(bits < cand, 1, 0)
        acc = lax.fori_loop(0, nchunks, chunk,
                            jnp.zeros((_LANES,), jnp.int32), unroll=8)
        return all_sum(acc)

    def prefix_incl(x):
        y = x
        for sh in (1, 2, 4, 8):
            g = lane_gather(y, jnp.maximum(lane - sh, 0))
            y = y + jnp.where(lane >= sh, g, 0)
        return y

    pltpu.sync_copy(noise_hbm.at[wid], col_v)

    def col_body(i, _):
        def bit_iter(bi, t):
            cand = t + (jnp.int32(1) << (jnp.int32(29) - bi))
            return jnp.where(count_below(i, cand) < kvec, cand, t)
        t = lax.fori_loop(0, 30, bit_iter, jnp.zeros((_LANES,), jnp.int32))

        n_tie_keep = kvec - count_below(i, t)

        def tie_chunk(j, seq):
            off = pl.multiple_of(j * _LANES, _LANES)
            bits = col_v[i, pl.ds(off, _LANES)]
            below = bits < t
            eq = bits == t
            eqi = jnp.where(eq, 1, 0)
            incl = prefix_incl(eqi)
            excl = incl - eqi
            keep = below | (eq & ((seq + excl) < n_tie_keep))
            mask_v[i, pl.ds(off, _LANES)] = jnp.where(keep, 1.0, 0.0).astype(
                jnp.float32)
            return seq + lane_gather(incl, jnp.full((_LANES,), _LANES - 1,
                                                    jnp.int32))
        lax.fori_loop(0, nchunks, tie_chunk, jnp.zeros((_LANES,), jnp.int32),
                      unroll=4)
        return 0

    lax.fori_loop(0, nvars, col_body, 0)
    pltpu.sync_copy(mask_v, out_hbm.at[wid])


def _apply_kernel(xb_ref, mask_ref, out_ref):
    m = mask_ref[...]
    mt = jnp.transpose(m, (0, 2, 1))
    out_ref[...] = xb_ref[...] * mt[:, :, :, None]


@jax.jit
def kernel(xb, noise):
    bs, L, nvars, D = xb.shape
    len_keep = int(L * (1 - _MASK_RATIO))

    noise_r = jnp.transpose(noise, (0, 2, 1))

    mesh = plsc.VectorSubcoreMesh(core_axis_name="c", subcore_axis_name="s")
    sc_mask = functools.partial(
        pl.kernel,
        mesh=mesh,
        out_type=jax.ShapeDtypeStruct((bs, nvars, L), jnp.float32),
        scratch_types=[
            pltpu.VMEM((nvars, L), jnp.int32),
            pltpu.VMEM((nvars, L), jnp.float32),
        ],
    )(functools.partial(_sc_mask_body, len_keep, L, nvars))
    mask = sc_mask(lax.bitcast_convert_type(noise_r, jnp.int32))

    bb = 2
    grid = (bs // bb,)
    out = pl.pallas_call(
        _apply_kernel,
        grid=grid,
        compiler_params=pltpu.CompilerParams(
            dimension_semantics=("parallel",)),
        in_specs=[
            pl.BlockSpec((bb, L, nvars, D), lambda b: (b, 0, 0, 0)),
            pl.BlockSpec((bb, nvars, L), lambda b: (b, 0, 0)),
        ],
        out_specs=pl.BlockSpec((bb, L, nvars, D), lambda b: (b, 0, 0, 0)),
        out_shape=jax.ShapeDtypeStruct((bs, L, nvars, D), jnp.float32),
    )(xb, mask)
    return out
